# baseline (speedup 1.0000x reference)
import functools

import jax
import jax.numpy as jnp
from jax.experimental import pallas as pl
from jax.experimental.pallas import tpu as pltpu

S, L, B, D = 4, 512, 64, 256
_INF = 3.4e38

LC = 64
JB = 8
SUB = 8
NI = L // LC
NBB = B // JB

PAIRS = ([(s, t) for s in range(S) for t in range(S)]
         + [(s, -1) for s in range(S)] + [(-1, t) for t in range(S)])
NP = len(PAIRS)


def _greedy_loss(d):
    rows = jax.lax.broadcasted_iota(jnp.int32, (S, S), 0)
    cols = jax.lax.broadcasted_iota(jnp.int32, (S, S), 1)
    flat_ids = rows * S + cols

    loss_b = jnp.float32(0.0)
    for _ in range(S):
        mval = jnp.min(d)
        idx = jnp.min(jnp.where(d == mval, flat_ids, S * S))
        r = idx // S
        c = idx - r * S
        loss_b = loss_b + mval
        d = jnp.where((rows == r) | (cols == c), _INF, d)
    return loss_b


def _minloss_body(p_ref, g_ref, o_ref, acc_ref):
    bb = pl.program_id(0)
    i = pl.program_id(1)

    @pl.when(i == 0)
    def _zero_acc():
        acc_ref[...] = jnp.zeros_like(acc_ref)

    for j in range(JB):
        def chunk_body(k, accs, j=j):
            base = pl.multiple_of(k * SUB, SUB)
            Pc = [p_ref[s, pl.ds(base, SUB), j * D:(j + 1) * D]
                  for s in range(S)]
            Gc = [g_ref[t, j, pl.ds(base, SUB), :] for t in range(S)]
            Ph = [(x[:, :128], x[:, 128:]) for x in Pc]
            Gh = [(x[:, :128], x[:, 128:]) for x in Gc]
            out = []
            for (s, t), a in zip(PAIRS, accs):
                x = Ph[s] if s >= 0 else Gh[t]
                y = Gh[t] if t >= 0 else Ph[s]
                out.append(a + x[0] * y[0] + x[1] * y[1])
            return tuple(out)

        accs0 = tuple(jnp.zeros((SUB, 128), jnp.float32) for _ in PAIRS)
        accs = jax.lax.fori_loop(0, LC // SUB, chunk_body, accs0, unroll=2)

        for k in range(NP):
            row = j * NP + k
            acc_ref[row] = acc_ref[row] + accs[k]

    @pl.when(jnp.logical_and(i == 0, bb == 0))
    def _init_out():
        o_ref[...] = jnp.zeros_like(o_ref)

    @pl.when(i == NI - 1)
    def _finalize():
        loss_blk = jnp.float32(0.0)
        for j in range(JB):
            sums = [jnp.sum(acc_ref[j * NP + k]) for k in range(NP)]
            cross = {PAIRS[k]: sums[k] for k in range(NP)}
            d2 = jnp.stack(
                [jnp.stack([cross[(s, -1)] + cross[(-1, t)]
                            - 2.0 * cross[(s, t)] for t in range(S)])
                 for s in range(S)]
            )
            d = jnp.sqrt(jnp.maximum(d2, 0.0))
            loss_blk = loss_blk + _greedy_loss(d)
        o_ref[...] = o_ref[...] + loss_blk


def kernel(predictions, ground_truths):
    pred_r = predictions.reshape(S, L, B * D)

    out = pl.pallas_call(
        _minloss_body,
        grid=(NBB, NI),
        in_specs=[
            pl.BlockSpec((S, LC, JB * D), lambda bb, i: (0, i, bb)),
            pl.BlockSpec((S, JB, LC, D), lambda bb, i: (0, bb, i, 0)),
        ],
        out_specs=pl.BlockSpec((1, 1), lambda bb, i: (0, 0)),
        out_shape=jax.ShapeDtypeStruct((1, 1), jnp.float32),
        scratch_shapes=[pltpu.VMEM((JB * NP, SUB, 128), jnp.float32)],
    )(pred_r, ground_truths)
    return out[0, 0]


# speedup vs baseline: 1.0261x; 1.0261x over previous
import functools

import jax
import jax.numpy as jnp
from jax.experimental import pallas as pl
from jax.experimental.pallas import tpu as pltpu

S, L, B, D = 4, 512, 64, 256
_INF = 3.4e38

LC = 64
JB = 8
SUB = 8
NI = L // LC
NBB = B // JB

PAIRS = ([(s, t) for s in range(S) for t in range(S)]
         + [(s, -1) for s in range(S)] + [(-1, t) for t in range(S)])
NP = len(PAIRS)


def _greedy_loss(d):
    rows = jax.lax.broadcasted_iota(jnp.int32, (S, S), 0)
    cols = jax.lax.broadcasted_iota(jnp.int32, (S, S), 1)
    flat_ids = rows * S + cols

    loss_b = jnp.float32(0.0)
    for _ in range(S):
        mval = jnp.min(d)
        idx = jnp.min(jnp.where(d == mval, flat_ids, S * S))
        r = idx // S
        c = idx - r * S
        loss_b = loss_b + mval
        d = jnp.where((rows == r) | (cols == c), _INF, d)
    return loss_b


def _minloss_body(p_ref, g_ref, o_ref, acc_ref):
    bb = pl.program_id(0)
    i = pl.program_id(1)

    @pl.when(i == 0)
    def _zero_acc():
        for r in range(JB * NP):
            acc_ref[r] = jnp.float32(0.0)

    for j in range(JB):
        Ps = [p_ref[s, :, j * D:(j + 1) * D] for s in range(S)]
        Gs = [g_ref[t, j] for t in range(S)]
        for k, (s, t) in enumerate(PAIRS):
            x = Ps[s] if s >= 0 else Gs[t]
            y = Gs[t] if t >= 0 else Ps[s]
            acc_ref[j * NP + k] = acc_ref[j * NP + k] + jnp.sum(x * y)

    @pl.when(jnp.logical_and(i == 0, bb == 0))
    def _init_out():
        o_ref[...] = jnp.zeros_like(o_ref)

    @pl.when(i == NI - 1)
    def _finalize():
        loss_blk = jnp.float32(0.0)
        for j in range(JB):
            sums = [acc_ref[j * NP + k] for k in range(NP)]
            cross = {PAIRS[k]: sums[k] for k in range(NP)}
            d2 = jnp.stack(
                [jnp.stack([cross[(s, -1)] + cross[(-1, t)]
                            - 2.0 * cross[(s, t)] for t in range(S)])
                 for s in range(S)]
            )
            d = jnp.sqrt(jnp.maximum(d2, 0.0))
            loss_blk = loss_blk + _greedy_loss(d)
        o_ref[...] = o_ref[...] + loss_blk


def kernel(predictions, ground_truths):
    pred_r = predictions.reshape(S, L, B * D)

    out = pl.pallas_call(
        _minloss_body,
        grid=(NBB, NI),
        in_specs=[
            pl.BlockSpec((S, LC, JB * D), lambda bb, i: (0, i, bb)),
            pl.BlockSpec((S, JB, LC, D), lambda bb, i: (0, bb, i, 0)),
        ],
        out_specs=pl.BlockSpec((1, 1), lambda bb, i: (0, 0)),
        out_shape=jax.ShapeDtypeStruct((1, 1), jnp.float32),
        scratch_shapes=[pltpu.SMEM((JB * NP,), jnp.float32)],
    )(pred_r, ground_truths)
    return out[0, 0]


# speedup vs baseline: 1.2051x; 1.1744x over previous
import jax
import jax.numpy as jnp
from jax.experimental import pallas as pl

S, L, B, D = 4, 512, 64, 256
_INF = 3.4e38


BB = 4


def _greedy_loss(d):
    rows = jax.lax.broadcasted_iota(jnp.int32, (S, S), 0)
    cols = jax.lax.broadcasted_iota(jnp.int32, (S, S), 1)
    flat_ids = rows * S + cols

    loss_b = jnp.float32(0.0)
    for _ in range(S):
        mval = jnp.min(d)
        idx = jnp.min(jnp.where(d == mval, flat_ids, S * S))
        r = idx // S
        c = idx - r * S
        loss_b = loss_b + mval
        d = jnp.where((rows == r) | (cols == c), _INF, d)
    return loss_b


def _minloss_body(p_ref, g_ref, o_ref):
    b = pl.program_id(0)

    Pblk = p_ref[...]
    Gblk = g_ref[...]

    loss_blk = jnp.float32(0.0)
    for j in range(BB):
        P = Pblk[:, :, j * D:(j + 1) * D]
        G = Gblk[:, j * L:(j + 1) * L, :]

        Ps = [P[s] for s in range(S)]
        Gs = [G[t] for t in range(S)]

        d2 = jnp.stack(
            [jnp.stack([jnp.sum((Ps[s] - Gs[t]) ** 2) for t in range(S)])
             for s in range(S)]
        )
        d = jnp.sqrt(jnp.maximum(d2, 0.0))
        loss_blk = loss_blk + _greedy_loss(d)

    @pl.when(b == 0)
    def _init():
        o_ref[...] = jnp.zeros_like(o_ref)

    o_ref[...] = o_ref[...] + loss_blk


def kernel(predictions, ground_truths):
    pred_r = predictions.reshape(S, L, B * D)
    gt_r = ground_truths.reshape(S, B * L, D)

    out = pl.pallas_call(
        _minloss_body,
        grid=(B // BB,),
        in_specs=[
            pl.BlockSpec((S, L, BB * D), lambda b: (0, 0, b)),
            pl.BlockSpec((S, BB * L, D), lambda b: (0, b, 0)),
        ],
        out_specs=pl.BlockSpec((1, 1), lambda b: (0, 0)),
        out_shape=jax.ShapeDtypeStruct((1, 1), jnp.float32),
    )(pred_r, gt_r)
    return out[0, 0]
